# baseline (speedup 1.0000x reference)
import functools
import jax
import jax.numpy as jnp
from jax import lax
from jax.experimental import pallas as pl
from jax.experimental.pallas import tpu as pltpu, tpu_sc as plsc

_NC, _NS = 2, 16
_NW = _NC * _NS


def kernel(x, embed_table):
    B, T, D = x.shape
    ROWS = B * T
    RPW = ROWS // _NW
    CH = 16
    NCH = RPW // CH
    CHW = CH * D

    xf = x.reshape(ROWS * D)
    ef = embed_table.reshape(-1)
    mesh = plsc.VectorSubcoreMesh(core_axis_name="c", subcore_axis_name="s")

    @functools.partial(
        pl.kernel,
        out_type=jax.ShapeDtypeStruct((ROWS * D,), jnp.float32),
        mesh=mesh,
        scratch_types=[
            pltpu.VMEM((2, CHW), jnp.float32),
            pltpu.VMEM((2, CHW), jnp.float32),
            pltpu.SemaphoreType.DMA,
            pltpu.SemaphoreType.DMA,
            pltpu.SemaphoreType.DMA,
            pltpu.SemaphoreType.DMA,
            pltpu.SemaphoreType.DMA,
            pltpu.SemaphoreType.DMA,
        ],
    )
    def k(x_hbm, e_hbm, o_hbm, xb, eb, gx0, gx1, ge0, ge1, so0, so1):
        gx = (gx0, gx1)
        ge = (ge0, ge1)
        so = (so0, so1)
        w = lax.axis_index("c") * _NS + lax.axis_index("s")
        x0 = w * (RPW * D)
        e0 = lax.rem(w * RPW, T) * D

        gathers = [None, None]
        scatters = [None, None]

        def issue_gathers(c):
            s = c % 2
            dx = pltpu.async_copy(
                x_hbm.at[pl.ds(x0 + c * CHW, CHW)], xb.at[s], gx[s]
            )
            de = pltpu.async_copy(
                e_hbm.at[pl.ds(e0 + c * CHW, CHW)], eb.at[s], ge[s]
            )
            gathers[s] = (dx, de)

        issue_gathers(0)
        for c in range(NCH):
            s = c % 2
            if c + 1 < NCH:
                if scatters[1 - s] is not None:
                    scatters[1 - s].wait()
                issue_gathers(c + 1)
            dx, de = gathers[s]
            dx.wait()
            de.wait()
            xs = xb.at[s]
            es = eb.at[s]

            @plsc.parallel_loop(0, CHW, step=16, unroll=8)
            def body(i):
                xs[pl.ds(i, 16)] = xs[pl.ds(i, 16)] + es[pl.ds(i, 16)]

            scatters[s] = pltpu.async_copy(
                xb.at[s], o_hbm.at[pl.ds(x0 + c * CHW, CHW)], so[s]
            )
        for d in scatters:
            if d is not None:
                d.wait()

    return k(xf, ef).reshape(B, T, D)
